# baseline (speedup 1.0000x reference)
import functools

import jax
import jax.numpy as jnp
from jax import lax
from jax.experimental import pallas as pl
from jax.experimental.pallas import tpu as pltpu
from jax.experimental.pallas import tpu_sc as plsc

N_NODES = 100000
N_GRAPHS = 512
D_FEAT = 128
HIDDEN = 256
D_TARGET = 64

NC = 2
NS = 16
NW = NC * NS
CHUNK = 200
NCHUNK = 9
ROWS_PER_W = CHUNK * NCHUNK
SC_ROWS = NW * ROWS_PER_W

TC_BLK = 800
TC_OFF = SC_ROWS // TC_BLK
TC_NBLK = (N_NODES - SC_ROWS) // TC_BLK


NBUF = 4
LAG = 2


def _sc_segsum_body(x_hbm, ids_hbm, zeros_hbm, part_hbm, *refs):
    idx_v = refs[0]
    bufs = refs[1:1 + NBUF]
    acc_sh = refs[1 + NBUF]
    sems_l = refs[2 + NBUF:2 + 2 * NBUF]
    sems_s = refs[2 + 2 * NBUF:2 + 3 * NBUF]

    c = lax.axis_index("c")
    s = lax.axis_index("s")
    wid = s * NC + c
    base = wid * ROWS_PER_W

    @pl.when(s == 0)
    def _zero():
        pltpu.sync_copy(zeros_hbm, acc_sh)

    pltpu.sync_copy(ids_hbm.at[pl.ds(base, ROWS_PER_W)], idx_v)

    load_h = [None] * NCHUNK
    scat_h = [None] * NCHUNK
    for i in range(min(NBUF, NCHUNK)):
        load_h[i] = pltpu.async_copy(
            x_hbm.at[pl.ds(base + i * CHUNK, CHUNK)], bufs[i], sems_l[i])

    plsc.subcore_barrier()

    waited = set()
    for k in range(NCHUNK):
        load_h[k].wait()
        scat_h[k] = pltpu.async_copy(
            bufs[k % NBUF], acc_sh.at[idx_v.at[pl.ds(k * CHUNK, CHUNK)]],
            sems_s[k % NBUF], add=True)
        nxt = k + NBUF - LAG
        if k >= LAG and nxt < NCHUNK:
            scat_h[k - LAG].wait()
            waited.add(k - LAG)
            load_h[nxt] = pltpu.async_copy(
                x_hbm.at[pl.ds(base + nxt * CHUNK, CHUNK)],
                bufs[nxt % NBUF], sems_l[nxt % NBUF])
    for k in range(NCHUNK):
        if k not in waited:
            scat_h[k].wait()

    plsc.subcore_barrier()

    @pl.when(s == 0)
    def _flush():
        pltpu.sync_copy(acc_sh, part_hbm.at[c])


def _sc_segsum(x, ids2d, zeros):
    mesh = plsc.VectorSubcoreMesh(core_axis_name="c", subcore_axis_name="s")
    f = pl.kernel(
        _sc_segsum_body,
        mesh=mesh,
        out_type=jax.ShapeDtypeStruct((NC, N_GRAPHS, D_FEAT), jnp.float32),
        scratch_types=(
            [pltpu.VMEM((ROWS_PER_W,), jnp.int32)]
            + [pltpu.VMEM((CHUNK, D_FEAT), jnp.float32) for _ in range(NBUF)]
            + [pltpu.VMEM_SHARED((N_GRAPHS, D_FEAT), jnp.float32)]
            + [pltpu.SemaphoreType.DMA for _ in range(2 * NBUF)]
        ),
        compiler_params=pltpu.CompilerParams(use_tc_tiling_on_sc=False),
        cost_estimate=pl.CostEstimate(
            flops=SC_ROWS * D_FEAT,
            bytes_accessed=2 * SC_ROWS * D_FEAT * 4,
            transcendentals=0,
        ),
    )
    return f(x, ids2d, zeros)


def _tc_segsum_body(batch_ref, x_ref, part_ref):
    i = pl.program_id(0)

    @pl.when(i == 0)
    def _init():
        part_ref[...] = jnp.zeros_like(part_ref)

    seg = lax.broadcasted_iota(jnp.int32, (N_GRAPHS, TC_BLK), 0)
    onehot = (seg == batch_ref[0]).astype(jnp.bfloat16)
    x_bf = x_ref[...].astype(jnp.bfloat16)
    part_ref[...] += jnp.dot(onehot, x_bf,
                             preferred_element_type=jnp.float32)


def _tc_segsum(x, batch3d):
    return pl.pallas_call(
        _tc_segsum_body,
        grid=(TC_NBLK,),
        in_specs=[
            pl.BlockSpec((1, 1, TC_BLK), lambda i: (i + TC_OFF, 0, 0)),
            pl.BlockSpec((TC_BLK, D_FEAT), lambda i: (i + TC_OFF, 0)),
        ],
        out_specs=pl.BlockSpec((N_GRAPHS, D_FEAT), lambda i: (0, 0)),
        out_shape=jax.ShapeDtypeStruct((N_GRAPHS, D_FEAT), jnp.float32),
        cost_estimate=pl.CostEstimate(
            flops=2 * N_GRAPHS * (N_NODES - SC_ROWS) * D_FEAT,
            bytes_accessed=(N_NODES - SC_ROWS) * D_FEAT * 4,
            transcendentals=0,
        ),
    )(batch3d, x)


def _mlp_body(scp_ref, tcp_ref, w1_ref, b1_ref, w2_ref, b2_ref, out_ref):
    pooled = scp_ref[0] + scp_ref[1] + tcp_ref[...]
    h = lax.dot_general(pooled, w1_ref[...], (((1,), (1,)), ((), ())),
                        preferred_element_type=jnp.float32)
    h = jnp.maximum(h + b1_ref[...], 0.0)
    o = lax.dot_general(h, w2_ref[...], (((1,), (1,)), ((), ())),
                        preferred_element_type=jnp.float32)
    out_ref[...] = o + b2_ref[...]


def _mlp(sc_part, tc_part, W1, b1, W2, b2):
    return pl.pallas_call(
        _mlp_body,
        out_shape=jax.ShapeDtypeStruct((N_GRAPHS, D_TARGET), jnp.float32),
    )(sc_part, tc_part, W1, b1.reshape(1, HIDDEN), W2, b2.reshape(1, D_TARGET))


def kernel(x, edge_index, batch, W1, b1, W2, b2):
    del edge_index
    batch = batch.astype(jnp.int32)
    batch3d = batch.reshape(N_NODES // TC_BLK, 1, TC_BLK)
    zeros = jnp.zeros((N_GRAPHS, D_FEAT), jnp.float32)
    tc_part = _tc_segsum(x, batch3d)
    sc_part = _sc_segsum(x, batch, zeros)
    return _mlp(sc_part, tc_part, W1, b1, W2, b2)


# speedup vs baseline: 1.3079x; 1.3079x over previous
import functools

import jax
import jax.numpy as jnp
from jax import lax
from jax.experimental import pallas as pl
from jax.experimental.pallas import tpu as pltpu
from jax.experimental.pallas import tpu_sc as plsc

N_NODES = 100000
N_GRAPHS = 512
D_FEAT = 128
HIDDEN = 256
D_TARGET = 64

NC = 2
NS = 16
NW = NC * NS
CHUNK = 200
NCHUNK = 10
ROWS_PER_W = CHUNK * NCHUNK
SC_ROWS = NW * ROWS_PER_W

TC_BLK = 2000
TC_OFF = SC_ROWS // TC_BLK
TC_NBLK = (N_NODES - SC_ROWS) // TC_BLK


NBUF = 4
LAG = 2


def _sc_segsum_body(x_hbm, ids_hbm, zeros_hbm, part_hbm, *refs):
    idx_v = refs[0]
    bufs = refs[1:1 + NBUF]
    acc_sh = refs[1 + NBUF]
    sems_l = refs[2 + NBUF:2 + 2 * NBUF]
    sems_s = refs[2 + 2 * NBUF:2 + 3 * NBUF]

    c = lax.axis_index("c")
    s = lax.axis_index("s")
    wid = s * NC + c
    base = wid * ROWS_PER_W

    @pl.when(s == 0)
    def _zero():
        pltpu.sync_copy(zeros_hbm, acc_sh)

    pltpu.sync_copy(ids_hbm.at[pl.ds(base, ROWS_PER_W)], idx_v)

    load_h = [None] * NCHUNK
    scat_h = [None] * NCHUNK
    for i in range(min(NBUF, NCHUNK)):
        load_h[i] = pltpu.async_copy(
            x_hbm.at[pl.ds(base + i * CHUNK, CHUNK)], bufs[i], sems_l[i])

    plsc.subcore_barrier()

    waited = set()
    for k in range(NCHUNK):
        load_h[k].wait()
        scat_h[k] = pltpu.async_copy(
            bufs[k % NBUF], acc_sh.at[idx_v.at[pl.ds(k * CHUNK, CHUNK)]],
            sems_s[k % NBUF], add=True)
        nxt = k + NBUF - LAG
        if k >= LAG and nxt < NCHUNK:
            scat_h[k - LAG].wait()
            waited.add(k - LAG)
            load_h[nxt] = pltpu.async_copy(
                x_hbm.at[pl.ds(base + nxt * CHUNK, CHUNK)],
                bufs[nxt % NBUF], sems_l[nxt % NBUF])
    for k in range(NCHUNK):
        if k not in waited:
            scat_h[k].wait()

    plsc.subcore_barrier()

    @pl.when(s == 0)
    def _flush():
        pltpu.sync_copy(acc_sh, part_hbm.at[c])


def _sc_segsum(x, ids2d, zeros):
    mesh = plsc.VectorSubcoreMesh(core_axis_name="c", subcore_axis_name="s")
    f = pl.kernel(
        _sc_segsum_body,
        mesh=mesh,
        out_type=jax.ShapeDtypeStruct((NC, N_GRAPHS, D_FEAT), jnp.float32),
        scratch_types=(
            [pltpu.VMEM((ROWS_PER_W,), jnp.int32)]
            + [pltpu.VMEM((CHUNK, D_FEAT), jnp.float32) for _ in range(NBUF)]
            + [pltpu.VMEM_SHARED((N_GRAPHS, D_FEAT), jnp.float32)]
            + [pltpu.SemaphoreType.DMA for _ in range(2 * NBUF)]
        ),
        compiler_params=pltpu.CompilerParams(use_tc_tiling_on_sc=False),
        cost_estimate=pl.CostEstimate(
            flops=SC_ROWS * D_FEAT,
            bytes_accessed=2 * SC_ROWS * D_FEAT * 4,
            transcendentals=0,
        ),
    )
    return f(x, ids2d, zeros)


def _tc_segsum_body(batch_ref, x_ref, part_ref):
    i = pl.program_id(0)

    @pl.when(i == 0)
    def _init():
        part_ref[...] = jnp.zeros_like(part_ref)

    seg = lax.broadcasted_iota(jnp.int32, (N_GRAPHS, TC_BLK), 0)
    onehot = (seg == batch_ref[0]).astype(jnp.bfloat16)
    x_bf = x_ref[...].astype(jnp.bfloat16)
    part_ref[...] += jnp.dot(onehot, x_bf,
                             preferred_element_type=jnp.float32)


def _tc_segsum(x, batch3d):
    return pl.pallas_call(
        _tc_segsum_body,
        grid=(TC_NBLK,),
        in_specs=[
            pl.BlockSpec((1, 1, TC_BLK), lambda i: (i + TC_OFF, 0, 0)),
            pl.BlockSpec((TC_BLK, D_FEAT), lambda i: (i + TC_OFF, 0)),
        ],
        out_specs=pl.BlockSpec((N_GRAPHS, D_FEAT), lambda i: (0, 0)),
        out_shape=jax.ShapeDtypeStruct((N_GRAPHS, D_FEAT), jnp.float32),
        cost_estimate=pl.CostEstimate(
            flops=2 * N_GRAPHS * (N_NODES - SC_ROWS) * D_FEAT,
            bytes_accessed=(N_NODES - SC_ROWS) * D_FEAT * 4,
            transcendentals=0,
        ),
    )(batch3d, x)


def _mlp_body(scp_ref, tcp_ref, w1_ref, b1_ref, w2_ref, b2_ref, out_ref):
    pooled = scp_ref[0] + scp_ref[1] + tcp_ref[...]
    h = lax.dot_general(pooled, w1_ref[...], (((1,), (1,)), ((), ())),
                        preferred_element_type=jnp.float32)
    h = jnp.maximum(h + b1_ref[...], 0.0)
    o = lax.dot_general(h, w2_ref[...], (((1,), (1,)), ((), ())),
                        preferred_element_type=jnp.float32)
    out_ref[...] = o + b2_ref[...]


def _mlp(sc_part, tc_part, W1, b1, W2, b2):
    return pl.pallas_call(
        _mlp_body,
        out_shape=jax.ShapeDtypeStruct((N_GRAPHS, D_TARGET), jnp.float32),
    )(sc_part, tc_part, W1, b1.reshape(1, HIDDEN), W2, b2.reshape(1, D_TARGET))


def kernel(x, edge_index, batch, W1, b1, W2, b2):
    del edge_index
    batch = batch.astype(jnp.int32)
    batch3d = batch.reshape(N_NODES // TC_BLK, 1, TC_BLK)
    zeros = jnp.zeros((N_GRAPHS, D_FEAT), jnp.float32)
    tc_part = _tc_segsum(x, batch3d)
    sc_part = _sc_segsum(x, batch, zeros)
    return _mlp(sc_part, tc_part, W1, b1, W2, b2)


# speedup vs baseline: 1.3580x; 1.0383x over previous
import functools

import jax
import jax.numpy as jnp
from jax import lax
from jax.experimental import pallas as pl
from jax.experimental.pallas import tpu as pltpu
from jax.experimental.pallas import tpu_sc as plsc

N_NODES = 100000
N_GRAPHS = 512
D_FEAT = 128
HIDDEN = 256
D_TARGET = 64

NC = 2
NS = 16
NW = NC * NS
CHUNK = 125
NCHUNK = 15
ROWS_PER_W = CHUNK * NCHUNK
SC_ROWS = NW * ROWS_PER_W

TC_BLK = 4000
TC_OFF = SC_ROWS // TC_BLK
TC_NBLK = (N_NODES - SC_ROWS) // TC_BLK


NBUF = 6
LAG = 2


def _sc_segsum_body(x_hbm, ids_hbm, zeros_hbm, part_hbm, *refs):
    idx_v = refs[0]
    bufs = refs[1:1 + NBUF]
    acc_sh = refs[1 + NBUF]
    sems_l = refs[2 + NBUF:2 + 2 * NBUF]
    sems_s = refs[2 + 2 * NBUF:2 + 3 * NBUF]

    c = lax.axis_index("c")
    s = lax.axis_index("s")
    wid = s * NC + c
    base = wid * ROWS_PER_W

    @pl.when(s == 0)
    def _zero():
        pltpu.sync_copy(zeros_hbm, acc_sh)

    pltpu.sync_copy(ids_hbm.at[pl.ds(wid * NCHUNK, NCHUNK)], idx_v)

    load_h = [None] * NCHUNK
    scat_h = [None] * NCHUNK
    for i in range(min(NBUF, NCHUNK)):
        load_h[i] = pltpu.async_copy(
            x_hbm.at[pl.ds(base + i * CHUNK, CHUNK)], bufs[i], sems_l[i])

    plsc.subcore_barrier()

    waited = set()
    for k in range(NCHUNK):
        load_h[k].wait()
        scat_h[k] = pltpu.async_copy(
            bufs[k % NBUF], acc_sh.at[idx_v.at[k]], sems_s[k % NBUF], add=True)
        nxt = k + NBUF - LAG
        if k >= LAG and nxt < NCHUNK:
            scat_h[k - LAG].wait()
            waited.add(k - LAG)
            load_h[nxt] = pltpu.async_copy(
                x_hbm.at[pl.ds(base + nxt * CHUNK, CHUNK)],
                bufs[nxt % NBUF], sems_l[nxt % NBUF])
    for k in range(NCHUNK):
        if k not in waited:
            scat_h[k].wait()

    plsc.subcore_barrier()

    @pl.when(s == 0)
    def _flush():
        pltpu.sync_copy(acc_sh, part_hbm.at[c])


def _sc_segsum(x, ids2d, zeros):
    mesh = plsc.VectorSubcoreMesh(core_axis_name="c", subcore_axis_name="s")
    f = pl.kernel(
        _sc_segsum_body,
        mesh=mesh,
        out_type=jax.ShapeDtypeStruct((NC, N_GRAPHS, D_FEAT), jnp.float32),
        scratch_types=(
            [pltpu.VMEM((NCHUNK, CHUNK), jnp.int32)]
            + [pltpu.VMEM((CHUNK, D_FEAT), jnp.float32) for _ in range(NBUF)]
            + [pltpu.VMEM_SHARED((N_GRAPHS, D_FEAT), jnp.float32)]
            + [pltpu.SemaphoreType.DMA for _ in range(2 * NBUF)]
        ),
        compiler_params=pltpu.CompilerParams(use_tc_tiling_on_sc=False),
        cost_estimate=pl.CostEstimate(
            flops=SC_ROWS * D_FEAT,
            bytes_accessed=2 * SC_ROWS * D_FEAT * 4,
            transcendentals=0,
        ),
    )
    return f(x, ids2d, zeros)


def _tc_segsum_body(batch_ref, x_ref, part_ref):
    i = pl.program_id(0)

    @pl.when(i == 0)
    def _init():
        part_ref[...] = jnp.zeros_like(part_ref)

    seg = lax.broadcasted_iota(jnp.int32, (N_GRAPHS, TC_BLK), 0)
    onehot = (seg == batch_ref[0]).astype(jnp.bfloat16)
    x_bf = x_ref[...].astype(jnp.bfloat16)
    part_ref[...] += jnp.dot(onehot, x_bf,
                             preferred_element_type=jnp.float32)


def _tc_segsum(x, batch3d):
    return pl.pallas_call(
        _tc_segsum_body,
        grid=(TC_NBLK,),
        in_specs=[
            pl.BlockSpec((1, 1, TC_BLK), lambda i: (i + TC_OFF, 0, 0)),
            pl.BlockSpec((TC_BLK, D_FEAT), lambda i: (i + TC_OFF, 0)),
        ],
        out_specs=pl.BlockSpec((N_GRAPHS, D_FEAT), lambda i: (0, 0)),
        out_shape=jax.ShapeDtypeStruct((N_GRAPHS, D_FEAT), jnp.float32),
        cost_estimate=pl.CostEstimate(
            flops=2 * N_GRAPHS * (N_NODES - SC_ROWS) * D_FEAT,
            bytes_accessed=(N_NODES - SC_ROWS) * D_FEAT * 4,
            transcendentals=0,
        ),
    )(batch3d, x)


def _mlp_body(scp_ref, tcp_ref, w1_ref, b1_ref, w2_ref, b2_ref, out_ref):
    pooled = scp_ref[0] + scp_ref[1] + tcp_ref[...]
    h = lax.dot_general(pooled, w1_ref[...], (((1,), (1,)), ((), ())),
                        preferred_element_type=jnp.float32)
    h = jnp.maximum(h + b1_ref[...], 0.0)
    o = lax.dot_general(h, w2_ref[...], (((1,), (1,)), ((), ())),
                        preferred_element_type=jnp.float32)
    out_ref[...] = o + b2_ref[...]


def _mlp(sc_part, tc_part, W1, b1, W2, b2):
    return pl.pallas_call(
        _mlp_body,
        out_shape=jax.ShapeDtypeStruct((N_GRAPHS, D_TARGET), jnp.float32),
    )(sc_part, tc_part, W1, b1.reshape(1, HIDDEN), W2, b2.reshape(1, D_TARGET))


def kernel(x, edge_index, batch, W1, b1, W2, b2):
    del edge_index
    batch = batch.astype(jnp.int32)
    ids2d = batch[:SC_ROWS].reshape(NW * NCHUNK, CHUNK)
    batch3d = batch.reshape(N_NODES // TC_BLK, 1, TC_BLK)
    zeros = jnp.zeros((N_GRAPHS, D_FEAT), jnp.float32)
    tc_part = _tc_segsum(x, batch3d)
    sc_part = _sc_segsum(x, ids2d, zeros)
    return _mlp(sc_part, tc_part, W1, b1, W2, b2)


# speedup vs baseline: 1.3603x; 1.0017x over previous
import functools

import jax
import jax.numpy as jnp
from jax import lax
from jax.experimental import pallas as pl
from jax.experimental.pallas import tpu as pltpu
from jax.experimental.pallas import tpu_sc as plsc

N_NODES = 100000
N_GRAPHS = 512
D_FEAT = 128
HIDDEN = 256
D_TARGET = 64

NC = 2
NS = 16
NW = NC * NS
CHUNK = 125
NCHUNK = 15
ROWS_PER_W = CHUNK * NCHUNK
SC_ROWS = NW * ROWS_PER_W

TC_BLK = 5000
TC_OFF = SC_ROWS // TC_BLK
TC_NBLK = (N_NODES - SC_ROWS) // TC_BLK


NBUF = 6
LAG = 2


def _sc_segsum_body(x_hbm, ids_hbm, zeros_hbm, part_hbm, *refs):
    idx_v = refs[0]
    bufs = refs[1:1 + NBUF]
    acc_sh = refs[1 + NBUF]
    sems_l = refs[2 + NBUF:2 + 2 * NBUF]
    sems_s = refs[2 + 2 * NBUF:2 + 3 * NBUF]

    c = lax.axis_index("c")
    s = lax.axis_index("s")
    wid = s * NC + c
    base = wid * ROWS_PER_W

    @pl.when(s == 0)
    def _zero():
        pltpu.sync_copy(zeros_hbm, acc_sh)

    pltpu.sync_copy(ids_hbm.at[pl.ds(wid * NCHUNK, NCHUNK)], idx_v)

    load_h = [None] * NCHUNK
    scat_h = [None] * NCHUNK
    for i in range(min(NBUF, NCHUNK)):
        load_h[i] = pltpu.async_copy(
            x_hbm.at[pl.ds(base + i * CHUNK, CHUNK)], bufs[i], sems_l[i])

    plsc.subcore_barrier()

    waited = set()
    for k in range(NCHUNK):
        load_h[k].wait()
        scat_h[k] = pltpu.async_copy(
            bufs[k % NBUF], acc_sh.at[idx_v.at[k]], sems_s[k % NBUF], add=True)
        nxt = k + NBUF - LAG
        if k >= LAG and nxt < NCHUNK:
            scat_h[k - LAG].wait()
            waited.add(k - LAG)
            load_h[nxt] = pltpu.async_copy(
                x_hbm.at[pl.ds(base + nxt * CHUNK, CHUNK)],
                bufs[nxt % NBUF], sems_l[nxt % NBUF])
    for k in range(NCHUNK):
        if k not in waited:
            scat_h[k].wait()

    plsc.subcore_barrier()

    @pl.when(s == 0)
    def _flush():
        pltpu.sync_copy(acc_sh, part_hbm.at[c])


def _sc_segsum(x, ids2d, zeros):
    mesh = plsc.VectorSubcoreMesh(core_axis_name="c", subcore_axis_name="s")
    f = pl.kernel(
        _sc_segsum_body,
        mesh=mesh,
        out_type=jax.ShapeDtypeStruct((NC, N_GRAPHS, D_FEAT), jnp.float32),
        scratch_types=(
            [pltpu.VMEM((NCHUNK, CHUNK), jnp.int32)]
            + [pltpu.VMEM((CHUNK, D_FEAT), jnp.float32) for _ in range(NBUF)]
            + [pltpu.VMEM_SHARED((N_GRAPHS, D_FEAT), jnp.float32)]
            + [pltpu.SemaphoreType.DMA for _ in range(2 * NBUF)]
        ),
        compiler_params=pltpu.CompilerParams(use_tc_tiling_on_sc=False),
        cost_estimate=pl.CostEstimate(
            flops=SC_ROWS * D_FEAT,
            bytes_accessed=2 * SC_ROWS * D_FEAT * 4,
            transcendentals=0,
        ),
    )
    return f(x, ids2d, zeros)


def _tc_segsum_body(batch_ref, x_ref, part_ref):
    i = pl.program_id(0)

    @pl.when(i == 0)
    def _init():
        part_ref[...] = jnp.zeros_like(part_ref)

    seg = lax.broadcasted_iota(jnp.int32, (N_GRAPHS, TC_BLK), 0)
    onehot = (seg == batch_ref[0]).astype(jnp.bfloat16)
    x_bf = x_ref[...].astype(jnp.bfloat16)
    part_ref[...] += jnp.dot(onehot, x_bf,
                             preferred_element_type=jnp.float32)


def _tc_segsum(x, batch3d):
    return pl.pallas_call(
        _tc_segsum_body,
        grid=(TC_NBLK,),
        in_specs=[
            pl.BlockSpec((1, 1, TC_BLK), lambda i: (i + TC_OFF, 0, 0)),
            pl.BlockSpec((TC_BLK, D_FEAT), lambda i: (i + TC_OFF, 0)),
        ],
        out_specs=pl.BlockSpec((N_GRAPHS, D_FEAT), lambda i: (0, 0)),
        out_shape=jax.ShapeDtypeStruct((N_GRAPHS, D_FEAT), jnp.float32),
        cost_estimate=pl.CostEstimate(
            flops=2 * N_GRAPHS * (N_NODES - SC_ROWS) * D_FEAT,
            bytes_accessed=(N_NODES - SC_ROWS) * D_FEAT * 4,
            transcendentals=0,
        ),
    )(batch3d, x)


def _mlp_body(scp_ref, tcp_ref, w1_ref, b1_ref, w2_ref, b2_ref, out_ref):
    pooled = scp_ref[0] + scp_ref[1] + tcp_ref[...]
    h = lax.dot_general(pooled, w1_ref[...], (((1,), (1,)), ((), ())),
                        preferred_element_type=jnp.float32)
    h = jnp.maximum(h + b1_ref[...], 0.0)
    o = lax.dot_general(h, w2_ref[...], (((1,), (1,)), ((), ())),
                        preferred_element_type=jnp.float32)
    out_ref[...] = o + b2_ref[...]


def _mlp(sc_part, tc_part, W1, b1, W2, b2):
    return pl.pallas_call(
        _mlp_body,
        out_shape=jax.ShapeDtypeStruct((N_GRAPHS, D_TARGET), jnp.float32),
    )(sc_part, tc_part, W1, b1.reshape(1, HIDDEN), W2, b2.reshape(1, D_TARGET))


def kernel(x, edge_index, batch, W1, b1, W2, b2):
    del edge_index
    batch = batch.astype(jnp.int32)
    ids2d = batch[:SC_ROWS].reshape(NW * NCHUNK, CHUNK)
    batch3d = batch.reshape(N_NODES // TC_BLK, 1, TC_BLK)
    zeros = jnp.zeros((N_GRAPHS, D_FEAT), jnp.float32)
    tc_part = _tc_segsum(x, batch3d)
    sc_part = _sc_segsum(x, ids2d, zeros)
    return _mlp(sc_part, tc_part, W1, b1, W2, b2)
